# Initial kernel scaffold; baseline (speedup 1.0000x reference)
import jax
import jax.numpy as jnp
from jax.experimental import pallas as pl


def kernel(x, edge_attr, enc_W0, enc_b0, enc_W1, enc_b1, msg_W0, msg_b0, msg_W1, msg_b1, msg_W2, msg_b2, upd_W0, upd_b0, upd_W1, upd_b1, upd_W2, upd_b2, gn_gamma, gn_beta, dec_W0, dec_b0, dec_W1, dec_b1, edge_index):
    raise NotImplementedError("write your pallas kernel here")


# speedup vs baseline: 1.4675x; 1.4675x over previous
import functools

import jax
import jax.numpy as jnp
import numpy as np
from jax import lax
from jax.experimental import pallas as pl
from jax.experimental.pallas import tpu as pltpu
from jax.experimental.pallas import tpu_sc as plsc

N = 10000
D = 128
G = 8
N_PAD = 10240
E = 320000
E_PAD = 327680
NC, NS = 2, 16
NW = NC * NS
GCH = (2 * E_PAD) // (NW * 128)
SCH = E_PAD // (NW * 128)
RPS = N_PAD // NS
EPS = 1e-5
_EBLK = 1024
_NBLK = 1024


def _relu(v):
    return jnp.maximum(v, 0.0)


def _row_spec(blk, width):
    return pl.BlockSpec((blk, width), lambda i: (i, 0))


def _full_spec(shape):
    nd = len(shape)
    return pl.BlockSpec(shape, lambda i: (0,) * nd)


def _tc_call(body, grid, in_specs, out_specs, out_shape):
    return pl.pallas_call(
        body,
        grid=grid,
        in_specs=in_specs,
        out_specs=out_specs,
        out_shape=out_shape,
        compiler_params=pltpu.CompilerParams(
            dimension_semantics=("parallel",)),
    )


def _encode_body(x_ref, w0_ref, b0_ref, w1_ref, b1_ref, o_ref):
    t = _relu(jnp.dot(x_ref[...], w0_ref[...],
                      preferred_element_type=jnp.float32) + b0_ref[...])
    o_ref[...] = _relu(jnp.dot(t, w1_ref[...],
                               preferred_element_type=jnp.float32) + b1_ref[...])


def _node_pre_body(h_ref, wd_ref, ws_ref, b0_ref, o_ref):
    h = h_ref[...]
    o_ref[0] = jnp.dot(h, wd_ref[...], preferred_element_type=jnp.float32) + b0_ref[...]
    o_ref[1] = jnp.dot(h, ws_ref[...], preferred_element_type=jnp.float32)


def _edge_body(g2_ref, ea_ref, w0e_ref, w1_ref, b1_ref, w2_ref, b2_ref, o_ref):
    g2 = g2_ref[...]
    ce = jnp.dot(ea_ref[...], w0e_ref[...], preferred_element_type=jnp.float32)
    m = _relu(g2[:, :D] + g2[:, D:] + ce)
    m = _relu(jnp.dot(m, w1_ref[...], preferred_element_type=jnp.float32) + b1_ref[...])
    o_ref[...] = jnp.dot(m, w2_ref[...], preferred_element_type=jnp.float32) + b2_ref[...]


def _update_body(h_ref, p_ref, uh_ref, ua_ref, b0_ref, u1_ref, b1_ref,
                 u2_ref, b2_ref, gp_ref, gam_ref, bet_ref, o_ref):
    h = h_ref[...]
    aggr = p_ref[0] + p_ref[1]
    u = _relu(jnp.dot(h, uh_ref[...], preferred_element_type=jnp.float32)
              + jnp.dot(aggr, ua_ref[...], preferred_element_type=jnp.float32)
              + b0_ref[...])
    u = _relu(jnp.dot(u, u1_ref[...], preferred_element_type=jnp.float32) + b1_ref[...])
    u = jnp.dot(u, u2_ref[...], preferred_element_type=jnp.float32) + b2_ref[...]
    a = _relu(u)
    gp = gp_ref[...]
    mu = jnp.dot(a, gp, preferred_element_type=jnp.float32, precision=lax.Precision.HIGHEST)
    xc = a - mu
    var = jnp.dot(xc * xc, gp, preferred_element_type=jnp.float32, precision=lax.Precision.HIGHEST)
    o_ref[...] = xc / jnp.sqrt(var + EPS) * gam_ref[...] + bet_ref[...]


def _decode_body(h_ref, w0_ref, b0_ref, w1_ref, b1_ref, o_ref):
    t = _relu(jnp.dot(h_ref[...], w0_ref[...],
                      preferred_element_type=jnp.float32) + b0_ref[...])
    o_ref[...] = jnp.dot(t, w1_ref[...], preferred_element_type=jnp.float32) + b1_ref[...]


def _sc_gather(idx2d, T):
    mesh = plsc.VectorSubcoreMesh(core_axis_name="c", subcore_axis_name="s")

    @functools.partial(
        pl.kernel, mesh=mesh,
        out_type=jax.ShapeDtypeStruct((2 * E_PAD, 128), jnp.float32),
        scratch_types=[
            pltpu.VMEM((GCH, 128), jnp.int32),
            pltpu.VMEM((128, 128), jnp.float32),
            pltpu.SemaphoreType.DMA,
        ],
    )
    def k(idx_hbm, t_hbm, out_hbm, idx_v, rows_v, sem):
        wid = lax.axis_index("s") * NC + lax.axis_index("c")
        base = wid * (GCH * 128)
        pltpu.sync_copy(idx_hbm.at[pl.ds(wid * GCH, GCH)], idx_v)

        def body(j, carry):
            pltpu.async_copy(t_hbm.at[idx_v.at[j]], rows_v, sem).wait()
            pltpu.sync_copy(rows_v, out_hbm.at[pl.ds(base + j * 128, 128)])
            return carry

        lax.fori_loop(0, GCH, body, 0)

    return k(idx2d, T)


def _sc_scatter(M, sidx, zeros):
    mesh = plsc.VectorSubcoreMesh(core_axis_name="c", subcore_axis_name="s")

    @functools.partial(
        pl.kernel, mesh=mesh,
        out_type=jax.ShapeDtypeStruct((NC, N_PAD, 128), jnp.float32),
        scratch_types=[
            pltpu.VMEM((SCH, 128), jnp.int32),
            pltpu.VMEM((128, 128), jnp.float32),
            pltpu.VMEM_SHARED((N_PAD, 128), jnp.float32),
            pltpu.SemaphoreType.DMA,
        ],
    )
    def k(m_hbm, idx_hbm, zero_hbm, out_hbm, idx_v, m_v, acc_sh, sem):
        c = lax.axis_index("c")
        s = lax.axis_index("s")
        wid = s * NC + c
        pltpu.sync_copy(zero_hbm, m_v)
        for j in range(RPS // 128):
            pltpu.sync_copy(m_v, acc_sh.at[pl.ds(s * RPS + j * 128, 128)])
        plsc.subcore_barrier()
        pltpu.sync_copy(idx_hbm.at[pl.ds(wid * SCH, SCH)], idx_v)

        def body(j, carry):
            pltpu.sync_copy(m_hbm.at[pl.ds((wid * SCH + j) * 128, 128)], m_v)
            pltpu.sync_copy(m_v, acc_sh.at[idx_v.at[j]], add=True)
            return carry

        lax.fori_loop(0, SCH, body, 0)
        plsc.subcore_barrier()
        for j in range(RPS // 128):
            pltpu.sync_copy(acc_sh.at[pl.ds(s * RPS + j * 128, 128)], m_v)
            pltpu.sync_copy(m_v, out_hbm.at[c, pl.ds(s * RPS + j * 128, 128)])

    return k(M, sidx, zeros)


def _group_proj():
    gidx = np.arange(D) // (D // G)
    same = (gidx[:, None] == gidx[None, :]).astype(np.float32)
    return jnp.asarray(same / (D // G))


def kernel(x, edge_attr, enc_W0, enc_b0, enc_W1, enc_b1,
           msg_W0, msg_b0, msg_W1, msg_b1, msg_W2, msg_b2,
           upd_W0, upd_b0, upd_W1, upd_b1, upd_W2, upd_b2,
           gn_gamma, gn_beta, dec_W0, dec_b0, dec_W1, dec_b1,
           edge_index):
    f32 = jnp.float32
    src = edge_index[0]
    dst = edge_index[1]
    pe = E_PAD - E
    dst_g = jnp.concatenate([dst, jnp.zeros((pe,), jnp.int32)])
    src_g = jnp.concatenate([src, jnp.zeros((pe,), jnp.int32)])
    idx2 = jnp.stack([dst_g, src_g + N_PAD], axis=1).reshape(
        (2 * E_PAD) // 128, 128)
    sidx = jnp.concatenate(
        [dst, jnp.full((pe,), N, jnp.int32)]).reshape(E_PAD // 128, 128)
    x_p = jnp.pad(x, ((0, N_PAD - N), (0, 0)))
    ea_p = jnp.pad(edge_attr, ((0, pe), (0, 0)))
    zeros128 = jnp.zeros((128, 128), f32)
    gproj = _group_proj()
    gamma = gn_gamma.reshape(1, D)
    beta = gn_beta.reshape(1, D)

    ngrid = N_PAD // _NBLK
    egrid = E_PAD // _EBLK

    h = _tc_call(
        _encode_body, (ngrid,),
        [_row_spec(_NBLK, D), _full_spec((D, D)), _full_spec((1, D)),
         _full_spec((D, D)), _full_spec((1, D))],
        _row_spec(_NBLK, D),
        jax.ShapeDtypeStruct((N_PAD, D), f32),
    )(x_p, enc_W0, enc_b0.reshape(1, D), enc_W1, enc_b1.reshape(1, D))

    for l in range(msg_W0.shape[0]):
        T = _tc_call(
            _node_pre_body, (ngrid,),
            [_row_spec(_NBLK, D), _full_spec((D, D)), _full_spec((D, D)),
             _full_spec((1, D))],
            pl.BlockSpec((2, _NBLK, D), lambda i: (0, i, 0)),
            jax.ShapeDtypeStruct((2, N_PAD, D), f32),
        )(h, msg_W0[l, :D], msg_W0[l, D:2 * D], msg_b0[l].reshape(1, D))

        G2 = _sc_gather(idx2, T.reshape(2 * N_PAD, D)).reshape(E_PAD, 2 * D)

        M = _tc_call(
            _edge_body, (egrid,),
            [_row_spec(_EBLK, 2 * D), _row_spec(_EBLK, 4), _full_spec((4, D)),
             _full_spec((D, D)), _full_spec((1, D)),
             _full_spec((D, D)), _full_spec((1, D))],
            _row_spec(_EBLK, D),
            jax.ShapeDtypeStruct((E_PAD, D), f32),
        )(G2, ea_p, msg_W0[l, 2 * D:], msg_W1[l], msg_b1[l].reshape(1, D),
          msg_W2[l], msg_b2[l].reshape(1, D))

        P = _sc_scatter(M, sidx, zeros128)

        h = _tc_call(
            _update_body, (ngrid,),
            [_row_spec(_NBLK, D), pl.BlockSpec((2, _NBLK, D), lambda i: (0, i, 0)),
             _full_spec((D, D)), _full_spec((D, D)), _full_spec((1, D)),
             _full_spec((D, D)), _full_spec((1, D)),
             _full_spec((D, D)), _full_spec((1, D)),
             _full_spec((D, D)), _full_spec((1, D)), _full_spec((1, D))],
            _row_spec(_NBLK, D),
            jax.ShapeDtypeStruct((N_PAD, D), f32),
        )(h, P, upd_W0[l, :D], upd_W0[l, D:], upd_b0[l].reshape(1, D),
          upd_W1[l], upd_b1[l].reshape(1, D), upd_W2[l], upd_b2[l].reshape(1, D),
          gproj, gamma, beta)

    y = _tc_call(
        _decode_body, (ngrid,),
        [_row_spec(_NBLK, D), _full_spec((D, D)), _full_spec((1, D)),
         _full_spec((D, 4)), _full_spec((1, 4))],
        _row_spec(_NBLK, 4),
        jax.ShapeDtypeStruct((N_PAD, 4), f32),
    )(h, dec_W0, dec_b0.reshape(1, D), dec_W1, dec_b1.reshape(1, 4))

    return y[:N]


# speedup vs baseline: 1.5769x; 1.0745x over previous
import functools

import jax
import jax.numpy as jnp
import numpy as np
from jax import lax
from jax.experimental import pallas as pl
from jax.experimental.pallas import tpu as pltpu
from jax.experimental.pallas import tpu_sc as plsc

N = 10000
D = 128
G = 8
N_PAD = 10240
E = 320000
E_PAD = 327680
NC, NS = 2, 16
NW = NC * NS
GCH = (2 * E_PAD) // (NW * 128)
SCH = E_PAD // (NW * 128)
RPS = N_PAD // NS
EPS = 1e-5
_EBLK = 1024
_NBLK = 1024


def _relu(v):
    return jnp.maximum(v, 0.0)


def _row_spec(blk, width):
    return pl.BlockSpec((blk, width), lambda i: (i, 0))


def _full_spec(shape):
    nd = len(shape)
    return pl.BlockSpec(shape, lambda i: (0,) * nd)


def _tc_call(body, grid, in_specs, out_specs, out_shape):
    return pl.pallas_call(
        body,
        grid=grid,
        in_specs=in_specs,
        out_specs=out_specs,
        out_shape=out_shape,
        compiler_params=pltpu.CompilerParams(
            dimension_semantics=("parallel",)),
    )


def _encode_body(x_ref, w0_ref, b0_ref, w1_ref, b1_ref, o_ref):
    t = _relu(jnp.dot(x_ref[...], w0_ref[...],
                      preferred_element_type=jnp.float32) + b0_ref[...])
    o_ref[...] = _relu(jnp.dot(t, w1_ref[...],
                               preferred_element_type=jnp.float32) + b1_ref[...])


def _node_pre_body(h_ref, wd_ref, ws_ref, b0_ref, o_ref):
    h = h_ref[...]
    o_ref[0] = jnp.dot(h, wd_ref[...], preferred_element_type=jnp.float32) + b0_ref[...]
    o_ref[1] = jnp.dot(h, ws_ref[...], preferred_element_type=jnp.float32)


def _edge_body(g2_ref, ea_ref, w0e_ref, w1_ref, b1_ref, w2_ref, b2_ref, o_ref):
    g2 = g2_ref[...]
    ce = jnp.dot(ea_ref[...], w0e_ref[...], preferred_element_type=jnp.float32)
    m = _relu(g2[:, :D] + g2[:, D:] + ce)
    m = _relu(jnp.dot(m, w1_ref[...], preferred_element_type=jnp.float32) + b1_ref[...])
    o_ref[...] = jnp.dot(m, w2_ref[...], preferred_element_type=jnp.float32) + b2_ref[...]


def _update_body(h_ref, p_ref, uh_ref, ua_ref, b0_ref, u1_ref, b1_ref,
                 u2_ref, b2_ref, gp_ref, gam_ref, bet_ref, o_ref):
    h = h_ref[...]
    aggr = p_ref[0] + p_ref[1]
    u = _relu(jnp.dot(h, uh_ref[...], preferred_element_type=jnp.float32)
              + jnp.dot(aggr, ua_ref[...], preferred_element_type=jnp.float32)
              + b0_ref[...])
    u = _relu(jnp.dot(u, u1_ref[...], preferred_element_type=jnp.float32) + b1_ref[...])
    u = jnp.dot(u, u2_ref[...], preferred_element_type=jnp.float32) + b2_ref[...]
    a = _relu(u)
    gp = gp_ref[...]
    mu = jnp.dot(a, gp, preferred_element_type=jnp.float32, precision=lax.Precision.HIGHEST)
    xc = a - mu
    var = jnp.dot(xc * xc, gp, preferred_element_type=jnp.float32, precision=lax.Precision.HIGHEST)
    o_ref[...] = xc / jnp.sqrt(var + EPS) * gam_ref[...] + bet_ref[...]


def _decode_body(h_ref, w0_ref, b0_ref, w1_ref, b1_ref, o_ref):
    t = _relu(jnp.dot(h_ref[...], w0_ref[...],
                      preferred_element_type=jnp.float32) + b0_ref[...])
    o_ref[...] = jnp.dot(t, w1_ref[...], preferred_element_type=jnp.float32) + b1_ref[...]


_GNB = 8
_GROW = 64
_GNCH = (2 * E_PAD) // (NW * _GROW)


def _sc_gather(idx2d, T):
    mesh = plsc.VectorSubcoreMesh(core_axis_name="c", subcore_axis_name="s")

    @functools.partial(
        pl.kernel, mesh=mesh,
        out_type=jax.ShapeDtypeStruct((2 * E_PAD, 128), jnp.float32),
        scratch_types=[
            pltpu.VMEM((_GNCH, _GROW), jnp.int32),
            pltpu.VMEM((_GNB, _GROW, 128), jnp.float32),
            pltpu.SemaphoreType.DMA((_GNB,)),
            pltpu.SemaphoreType.DMA((_GNB,)),
        ],
    )
    def k(idx_hbm, t_hbm, out_hbm, idx_v, bufs, gsem, wsem):
        wid = lax.axis_index("s") * NC + lax.axis_index("c")
        rbase = wid * (_GNCH * _GROW)
        pltpu.sync_copy(idx_hbm.at[pl.ds(wid * _GNCH, _GNCH)], idx_v)

        def body(i, carry):
            base = i * _GNB
            for b in range(_GNB):
                kk = base + b

                @pl.when(i > 0)
                def _(b=b, kk=kk):
                    pltpu.make_async_copy(
                        bufs.at[b],
                        out_hbm.at[pl.ds(rbase + (kk - _GNB) * _GROW, _GROW)],
                        wsem.at[b]).wait()

                pltpu.async_copy(t_hbm.at[idx_v.at[kk]], bufs.at[b], gsem.at[b])
            for b in range(_GNB):
                kk = base + b
                pltpu.make_async_copy(t_hbm.at[idx_v.at[kk]], bufs.at[b],
                                      gsem.at[b]).wait()
                pltpu.async_copy(bufs.at[b],
                                 out_hbm.at[pl.ds(rbase + kk * _GROW, _GROW)],
                                 wsem.at[b])
            return carry

        lax.fori_loop(0, _GNCH // _GNB, body, 0)
        for b in range(_GNB):
            kk = _GNCH - _GNB + b
            pltpu.make_async_copy(
                bufs.at[b],
                out_hbm.at[pl.ds(rbase + kk * _GROW, _GROW)],
                wsem.at[b]).wait()

    return k(idx2d, T)


_SNB = 2


def _sc_scatter(M, sidx, zeros):
    mesh = plsc.VectorSubcoreMesh(core_axis_name="c", subcore_axis_name="s")

    @functools.partial(
        pl.kernel, mesh=mesh,
        out_type=jax.ShapeDtypeStruct((NC, N_PAD, 128), jnp.float32),
        scratch_types=[
            pltpu.VMEM((SCH, 128), jnp.int32),
            pltpu.VMEM((_SNB, 128, 128), jnp.float32),
            pltpu.VMEM_SHARED((N_PAD, 128), jnp.float32),
            pltpu.SemaphoreType.DMA((_SNB,)),
            pltpu.SemaphoreType.DMA((_SNB,)),
        ],
    )
    def k(m_hbm, idx_hbm, zero_hbm, out_hbm, idx_v, bufs, acc_sh, rsem, ssem):
        c = lax.axis_index("c")
        s = lax.axis_index("s")
        wid = s * NC + c
        m_v = bufs.at[0]
        pltpu.sync_copy(zero_hbm, m_v)
        for j in range(RPS // 128):
            pltpu.async_copy(m_v, acc_sh.at[pl.ds(s * RPS + j * 128, 128)],
                             rsem.at[0])
        for j in range(RPS // 128):
            pltpu.make_async_copy(
                m_v, acc_sh.at[pl.ds(s * RPS + j * 128, 128)], rsem.at[0]).wait()
        plsc.subcore_barrier()
        pltpu.sync_copy(idx_hbm.at[pl.ds(wid * SCH, SCH)], idx_v)

        def body(i, carry):
            base = i * _SNB
            for b in range(_SNB):
                j = base + b

                @pl.when(i > 0)
                def _(b=b, j=j):
                    pltpu.make_async_copy(
                        bufs.at[b], acc_sh.at[idx_v.at[j - _SNB]],
                        ssem.at[b]).wait()

                pltpu.async_copy(
                    m_hbm.at[pl.ds((wid * SCH + j) * 128, 128)], bufs.at[b],
                    rsem.at[b])
            for b in range(_SNB):
                j = base + b
                pltpu.make_async_copy(
                    m_hbm.at[pl.ds((wid * SCH + j) * 128, 128)], bufs.at[b],
                    rsem.at[b]).wait()
                pltpu.async_copy(bufs.at[b], acc_sh.at[idx_v.at[j]],
                                 ssem.at[b], add=True)
            return carry

        lax.fori_loop(0, SCH // _SNB, body, 0)
        for b in range(_SNB):
            j = SCH - _SNB + b
            pltpu.make_async_copy(bufs.at[b], acc_sh.at[idx_v.at[j]],
                                  ssem.at[b]).wait()
        plsc.subcore_barrier()
        m_v = bufs.at[0]
        for j in range(RPS // 128):
            pltpu.sync_copy(acc_sh.at[pl.ds(s * RPS + j * 128, 128)], m_v)
            pltpu.sync_copy(m_v, out_hbm.at[c, pl.ds(s * RPS + j * 128, 128)])

    return k(M, sidx, zeros)


def _group_proj():
    gidx = np.arange(D) // (D // G)
    same = (gidx[:, None] == gidx[None, :]).astype(np.float32)
    return jnp.asarray(same / (D // G))


def kernel(x, edge_attr, enc_W0, enc_b0, enc_W1, enc_b1,
           msg_W0, msg_b0, msg_W1, msg_b1, msg_W2, msg_b2,
           upd_W0, upd_b0, upd_W1, upd_b1, upd_W2, upd_b2,
           gn_gamma, gn_beta, dec_W0, dec_b0, dec_W1, dec_b1,
           edge_index):
    f32 = jnp.float32
    src = edge_index[0]
    dst = edge_index[1]
    pe = E_PAD - E
    dst_g = jnp.concatenate([dst, jnp.zeros((pe,), jnp.int32)])
    src_g = jnp.concatenate([src, jnp.zeros((pe,), jnp.int32)])
    idx2 = jnp.stack([dst_g, src_g + N_PAD], axis=1).reshape(
        (2 * E_PAD) // _GROW, _GROW)
    sidx = jnp.concatenate(
        [dst, jnp.full((pe,), N, jnp.int32)]).reshape(E_PAD // 128, 128)
    x_p = jnp.pad(x, ((0, N_PAD - N), (0, 0)))
    ea_p = jnp.pad(edge_attr, ((0, pe), (0, 0)))
    zeros128 = jnp.zeros((128, 128), f32)
    gproj = _group_proj()
    gamma = gn_gamma.reshape(1, D)
    beta = gn_beta.reshape(1, D)

    ngrid = N_PAD // _NBLK
    egrid = E_PAD // _EBLK

    h = _tc_call(
        _encode_body, (ngrid,),
        [_row_spec(_NBLK, D), _full_spec((D, D)), _full_spec((1, D)),
         _full_spec((D, D)), _full_spec((1, D))],
        _row_spec(_NBLK, D),
        jax.ShapeDtypeStruct((N_PAD, D), f32),
    )(x_p, enc_W0, enc_b0.reshape(1, D), enc_W1, enc_b1.reshape(1, D))

    for l in range(msg_W0.shape[0]):
        T = _tc_call(
            _node_pre_body, (ngrid,),
            [_row_spec(_NBLK, D), _full_spec((D, D)), _full_spec((D, D)),
             _full_spec((1, D))],
            pl.BlockSpec((2, _NBLK, D), lambda i: (0, i, 0)),
            jax.ShapeDtypeStruct((2, N_PAD, D), f32),
        )(h, msg_W0[l, :D], msg_W0[l, D:2 * D], msg_b0[l].reshape(1, D))

        G2 = _sc_gather(idx2, T.reshape(2 * N_PAD, D)).reshape(E_PAD, 2 * D)

        M = _tc_call(
            _edge_body, (egrid,),
            [_row_spec(_EBLK, 2 * D), _row_spec(_EBLK, 4), _full_spec((4, D)),
             _full_spec((D, D)), _full_spec((1, D)),
             _full_spec((D, D)), _full_spec((1, D))],
            _row_spec(_EBLK, D),
            jax.ShapeDtypeStruct((E_PAD, D), f32),
        )(G2, ea_p, msg_W0[l, 2 * D:], msg_W1[l], msg_b1[l].reshape(1, D),
          msg_W2[l], msg_b2[l].reshape(1, D))

        P = _sc_scatter(M, sidx, zeros128)

        h = _tc_call(
            _update_body, (ngrid,),
            [_row_spec(_NBLK, D), pl.BlockSpec((2, _NBLK, D), lambda i: (0, i, 0)),
             _full_spec((D, D)), _full_spec((D, D)), _full_spec((1, D)),
             _full_spec((D, D)), _full_spec((1, D)),
             _full_spec((D, D)), _full_spec((1, D)),
             _full_spec((D, D)), _full_spec((1, D)), _full_spec((1, D))],
            _row_spec(_NBLK, D),
            jax.ShapeDtypeStruct((N_PAD, D), f32),
        )(h, P, upd_W0[l, :D], upd_W0[l, D:], upd_b0[l].reshape(1, D),
          upd_W1[l], upd_b1[l].reshape(1, D), upd_W2[l], upd_b2[l].reshape(1, D),
          gproj, gamma, beta)

    y = _tc_call(
        _decode_body, (ngrid,),
        [_row_spec(_NBLK, D), _full_spec((D, D)), _full_spec((1, D)),
         _full_spec((D, 4)), _full_spec((1, 4))],
        _row_spec(_NBLK, 4),
        jax.ShapeDtypeStruct((N_PAD, 4), f32),
    )(h, dec_W0, dec_b0.reshape(1, D), dec_W1, dec_b1.reshape(1, 4))

    return y[:N]


# speedup vs baseline: 1.8636x; 1.1818x over previous
import functools

import jax
import jax.numpy as jnp
import numpy as np
from jax import lax
from jax.experimental import pallas as pl
from jax.experimental.pallas import tpu as pltpu
from jax.experimental.pallas import tpu_sc as plsc

N = 10000
D = 128
G = 8
N_PAD = 10240
E = 320000
E_PAD = 327680
NC, NS = 2, 16
NW = NC * NS
GCH = (2 * E_PAD) // (NW * 128)
SCH = E_PAD // (NW * 128)
RPS = N_PAD // NS
EPS = 1e-5
_EBLK = 1024
_NBLK = 1024


def _relu(v):
    return jnp.maximum(v, 0.0)


def _row_spec(blk, width):
    return pl.BlockSpec((blk, width), lambda i: (i, 0))


def _full_spec(shape):
    nd = len(shape)
    return pl.BlockSpec(shape, lambda i: (0,) * nd)


def _tc_call(body, grid, in_specs, out_specs, out_shape):
    return pl.pallas_call(
        body,
        grid=grid,
        in_specs=in_specs,
        out_specs=out_specs,
        out_shape=out_shape,
        compiler_params=pltpu.CompilerParams(
            dimension_semantics=("parallel",)),
    )


def _encode_body(x_ref, w0_ref, b0_ref, w1_ref, b1_ref, o_ref):
    t = _relu(jnp.dot(x_ref[...], w0_ref[...],
                      preferred_element_type=jnp.float32) + b0_ref[...])
    o_ref[...] = _relu(jnp.dot(t, w1_ref[...],
                               preferred_element_type=jnp.float32) + b1_ref[...])


def _node_pre_body(h_ref, wd_ref, ws_ref, b0_ref, o_ref):
    h = h_ref[...]
    o_ref[0] = (jnp.dot(h, wd_ref[...], preferred_element_type=jnp.float32)
                + b0_ref[...])
    o_ref[1] = jnp.dot(h, ws_ref[...], preferred_element_type=jnp.float32)


def _edge_body(a_ref, b_ref, ea_ref, w0e_ref, w1_ref, b1_ref, w2_ref, b2_ref,
               o_ref):
    ce = jnp.dot(ea_ref[...], w0e_ref[...], preferred_element_type=jnp.float32)
    m = _relu(a_ref[...] + b_ref[...] + ce)
    m = _relu(jnp.dot(m, w1_ref[...], preferred_element_type=jnp.float32) + b1_ref[...])
    o_ref[...] = jnp.dot(m, w2_ref[...], preferred_element_type=jnp.float32) + b2_ref[...]


def _update_body(h_ref, p_ref, uh_ref, ua_ref, b0_ref, u1_ref, b1_ref,
                 u2_ref, b2_ref, gp_ref, gam_ref, bet_ref, o_ref):
    h = h_ref[...]
    aggr = p_ref[0] + p_ref[1]
    u = _relu(jnp.dot(h, uh_ref[...], preferred_element_type=jnp.float32)
              + jnp.dot(aggr, ua_ref[...], preferred_element_type=jnp.float32)
              + b0_ref[...])
    u = _relu(jnp.dot(u, u1_ref[...], preferred_element_type=jnp.float32) + b1_ref[...])
    u = jnp.dot(u, u2_ref[...], preferred_element_type=jnp.float32) + b2_ref[...]
    a = _relu(u)
    gp = gp_ref[...]
    mu = jnp.dot(a, gp, preferred_element_type=jnp.float32, precision=lax.Precision.HIGHEST)
    xc = a - mu
    var = jnp.dot(xc * xc, gp, preferred_element_type=jnp.float32, precision=lax.Precision.HIGHEST)
    o_ref[...] = xc / jnp.sqrt(var + EPS) * gam_ref[...] + bet_ref[...]


def _decode_body(h_ref, w0_ref, b0_ref, w1_ref, b1_ref, o_ref):
    t = _relu(jnp.dot(h_ref[...], w0_ref[...],
                      preferred_element_type=jnp.float32) + b0_ref[...])
    o_ref[...] = jnp.dot(t, w1_ref[...], preferred_element_type=jnp.float32) + b1_ref[...]


_GNB = 4
_GROW = 128
_GNCH = (2 * E_PAD) // (NW * _GROW)


def _sc_gather(idx2d, T):
    mesh = plsc.VectorSubcoreMesh(core_axis_name="c", subcore_axis_name="s")

    @functools.partial(
        pl.kernel, mesh=mesh,
        out_type=jax.ShapeDtypeStruct((2 * E_PAD, 128), jnp.float32),
        scratch_types=[
            pltpu.VMEM((_GNCH, _GROW), jnp.int32),
            pltpu.VMEM((_GNB, _GROW, 128), jnp.float32),
            pltpu.SemaphoreType.DMA((_GNB,)),
            pltpu.SemaphoreType.DMA((_GNB,)),
        ],
    )
    def k(idx_hbm, t_hbm, out_hbm, idx_v, bufs, gsem, wsem):
        wid = lax.axis_index("s") * NC + lax.axis_index("c")
        rbase = wid * (_GNCH * _GROW)
        pltpu.sync_copy(idx_hbm.at[pl.ds(wid * _GNCH, _GNCH)], idx_v)

        def body(i, carry):
            base = i * _GNB
            for b in range(_GNB):
                kk = base + b

                @pl.when(i > 0)
                def _(b=b, kk=kk):
                    pltpu.make_async_copy(
                        bufs.at[b],
                        out_hbm.at[pl.ds(rbase + (kk - _GNB) * _GROW, _GROW)],
                        wsem.at[b]).wait()

                pltpu.async_copy(t_hbm.at[idx_v.at[kk]], bufs.at[b], gsem.at[b])
            for b in range(_GNB):
                kk = base + b
                pltpu.make_async_copy(t_hbm.at[idx_v.at[kk]], bufs.at[b],
                                      gsem.at[b]).wait()
                pltpu.async_copy(bufs.at[b],
                                 out_hbm.at[pl.ds(rbase + kk * _GROW, _GROW)],
                                 wsem.at[b])
            return carry

        lax.fori_loop(0, _GNCH // _GNB, body, 0)
        for b in range(_GNB):
            kk = _GNCH - _GNB + b
            pltpu.make_async_copy(
                bufs.at[b],
                out_hbm.at[pl.ds(rbase + kk * _GROW, _GROW)],
                wsem.at[b]).wait()

    return k(idx2d, T)


_SNB = 2


def _sc_scatter(M, sidx, zeros):
    mesh = plsc.VectorSubcoreMesh(core_axis_name="c", subcore_axis_name="s")

    @functools.partial(
        pl.kernel, mesh=mesh,
        out_type=jax.ShapeDtypeStruct((NC, N_PAD, 128), jnp.float32),
        scratch_types=[
            pltpu.VMEM((SCH, 128), jnp.int32),
            pltpu.VMEM((_SNB, 128, 128), jnp.float32),
            pltpu.VMEM_SHARED((N_PAD, 128), jnp.float32),
            pltpu.SemaphoreType.DMA((_SNB,)),
            pltpu.SemaphoreType.DMA((_SNB,)),
        ],
    )
    def k(m_hbm, idx_hbm, zero_hbm, out_hbm, idx_v, bufs, acc_sh, rsem, ssem):
        c = lax.axis_index("c")
        s = lax.axis_index("s")
        wid = s * NC + c
        m_v = bufs.at[0]
        pltpu.sync_copy(zero_hbm, m_v)
        for j in range(RPS // 128):
            pltpu.async_copy(m_v, acc_sh.at[pl.ds(s * RPS + j * 128, 128)],
                             rsem.at[0])
        for j in range(RPS // 128):
            pltpu.make_async_copy(
                m_v, acc_sh.at[pl.ds(s * RPS + j * 128, 128)], rsem.at[0]).wait()
        plsc.subcore_barrier()
        pltpu.sync_copy(idx_hbm.at[pl.ds(wid * SCH, SCH)], idx_v)

        def body(i, carry):
            base = i * _SNB
            for b in range(_SNB):
                j = base + b

                @pl.when(i > 0)
                def _(b=b, j=j):
                    pltpu.make_async_copy(
                        bufs.at[b], acc_sh.at[idx_v.at[j - _SNB]],
                        ssem.at[b]).wait()

                pltpu.async_copy(
                    m_hbm.at[pl.ds((wid * SCH + j) * 128, 128)], bufs.at[b],
                    rsem.at[b])
            for b in range(_SNB):
                j = base + b
                pltpu.make_async_copy(
                    m_hbm.at[pl.ds((wid * SCH + j) * 128, 128)], bufs.at[b],
                    rsem.at[b]).wait()
                pltpu.async_copy(bufs.at[b], acc_sh.at[idx_v.at[j]],
                                 ssem.at[b], add=True)
            return carry

        lax.fori_loop(0, SCH // _SNB, body, 0)
        for b in range(_SNB):
            j = SCH - _SNB + b
            pltpu.make_async_copy(bufs.at[b], acc_sh.at[idx_v.at[j]],
                                  ssem.at[b]).wait()
        plsc.subcore_barrier()
        m_v = bufs.at[0]
        for j in range(RPS // 128):
            pltpu.sync_copy(acc_sh.at[pl.ds(s * RPS + j * 128, 128)], m_v)
            pltpu.sync_copy(m_v, out_hbm.at[c, pl.ds(s * RPS + j * 128, 128)])

    return k(M, sidx, zeros)


def _group_proj():
    gidx = np.arange(D) // (D // G)
    same = (gidx[:, None] == gidx[None, :]).astype(np.float32)
    return jnp.asarray(same / (D // G))


def kernel(x, edge_attr, enc_W0, enc_b0, enc_W1, enc_b1,
           msg_W0, msg_b0, msg_W1, msg_b1, msg_W2, msg_b2,
           upd_W0, upd_b0, upd_W1, upd_b1, upd_W2, upd_b2,
           gn_gamma, gn_beta, dec_W0, dec_b0, dec_W1, dec_b1,
           edge_index):
    f32 = jnp.float32
    src = edge_index[0]
    dst = edge_index[1]
    pe = E_PAD - E
    dst_g = jnp.concatenate([dst, jnp.zeros((pe,), jnp.int32)])
    src_g = jnp.concatenate([src, jnp.zeros((pe,), jnp.int32)])
    idx2 = jnp.concatenate([dst_g, src_g + N_PAD]).reshape(
        (2 * E_PAD) // _GROW, _GROW)
    sidx = jnp.concatenate(
        [dst, jnp.full((pe,), N, jnp.int32)]).reshape(E_PAD // 128, 128)
    x_p = jnp.pad(x, ((0, N_PAD - N), (0, 0)))
    ea_p = jnp.pad(edge_attr, ((0, pe), (0, 0)))
    zeros128 = jnp.zeros((128, 128), f32)
    gproj = _group_proj()
    gamma = gn_gamma.reshape(1, D)
    beta = gn_beta.reshape(1, D)

    ngrid = N_PAD // _NBLK
    egrid = E_PAD // _EBLK

    h = _tc_call(
        _encode_body, (ngrid,),
        [_row_spec(_NBLK, D), _full_spec((D, D)), _full_spec((1, D)),
         _full_spec((D, D)), _full_spec((1, D))],
        _row_spec(_NBLK, D),
        jax.ShapeDtypeStruct((N_PAD, D), f32),
    )(x_p, enc_W0, enc_b0.reshape(1, D), enc_W1, enc_b1.reshape(1, D))

    for l in range(msg_W0.shape[0]):
        T = _tc_call(
            _node_pre_body, (ngrid,),
            [_row_spec(_NBLK, D), _full_spec((D, D)), _full_spec((D, D)),
             _full_spec((1, D))],
            pl.BlockSpec((2, _NBLK, D), lambda i: (0, i, 0)),
            jax.ShapeDtypeStruct((2, N_PAD, D), f32),
        )(h, msg_W0[l, :D], msg_W0[l, D:2 * D], msg_b0[l].reshape(1, D))

        G2 = _sc_gather(idx2, T.reshape(2 * N_PAD, D))

        M = _tc_call(
            _edge_body, (egrid,),
            [_row_spec(_EBLK, D),
             pl.BlockSpec((_EBLK, D), lambda i: (i + E_PAD // _EBLK, 0)),
             _row_spec(_EBLK, 4), _full_spec((4, D)),
             _full_spec((D, D)), _full_spec((1, D)),
             _full_spec((D, D)), _full_spec((1, D))],
            _row_spec(_EBLK, D),
            jax.ShapeDtypeStruct((E_PAD, D), f32),
        )(G2, G2, ea_p, msg_W0[l, 2 * D:], msg_W1[l],
          msg_b1[l].reshape(1, D), msg_W2[l], msg_b2[l].reshape(1, D))

        P = _sc_scatter(M, sidx, zeros128)

        h = _tc_call(
            _update_body, (ngrid,),
            [_row_spec(_NBLK, D), pl.BlockSpec((2, _NBLK, D), lambda i: (0, i, 0)),
             _full_spec((D, D)), _full_spec((D, D)), _full_spec((1, D)),
             _full_spec((D, D)), _full_spec((1, D)),
             _full_spec((D, D)), _full_spec((1, D)),
             _full_spec((D, D)), _full_spec((1, D)), _full_spec((1, D))],
            _row_spec(_NBLK, D),
            jax.ShapeDtypeStruct((N_PAD, D), f32),
        )(h, P, upd_W0[l, :D], upd_W0[l, D:], upd_b0[l].reshape(1, D),
          upd_W1[l], upd_b1[l].reshape(1, D), upd_W2[l], upd_b2[l].reshape(1, D),
          gproj, gamma, beta)

    y = _tc_call(
        _decode_body, (ngrid,),
        [_row_spec(_NBLK, D), _full_spec((D, D)), _full_spec((1, D)),
         _full_spec((D, 4)), _full_spec((1, 4))],
        _row_spec(_NBLK, 4),
        jax.ShapeDtypeStruct((N_PAD, 4), f32),
    )(h, dec_W0, dec_b0.reshape(1, D), dec_W1, dec_b1.reshape(1, 4))

    return y[:N]
